# Initial kernel scaffold; baseline (speedup 1.0000x reference)
import jax
import jax.numpy as jnp
from jax.experimental import pallas as pl


def kernel(atom_features, edges_sph_features, state_attrs, pair_indices, atom_graph_indices, bond_graph_indices, kernel_s, bias_s, kernel_g, bias_g):
    raise NotImplementedError("write your pallas kernel here")


# speedup vs baseline: 3.0440x; 3.0440x over previous
import functools

import jax
import jax.numpy as jnp
from jax import lax
from jax.experimental import pallas as pl
from jax.experimental.pallas import tpu as pltpu
from jax.experimental.pallas import tpu_sc as plsc

N_NODES = 10000
N_EDGES = 320000
D = 128
EDGE_DIM = 16
SG = 2 * D

NC = 2
NS = 16
NW = NC * NS
EW = N_EDGES // NW
CHUNK = 40
N_CHUNKS = EW // CHUNK
ROWS_PER_TILE = 624
TAIL_ROW0 = ROWS_PER_TILE * NS
TAIL_ROWS = N_NODES - TAIL_ROW0


def _softplus(x):
    return jnp.maximum(x, 0.0) + jnp.log(1.0 + jnp.exp(-jnp.abs(x)))


def _sigmoid(x):
    return 1.0 / (1.0 + jnp.exp(-x))


_R_TAB = 1000
_R_EDGE = 2000


def _tables_body(af_ref, w0_ref, w1_ref, b_ref, t0_ref, t1_ref):
    af = af_ref[...]
    t0_ref[...] = jnp.dot(af, w0_ref[...], preferred_element_type=jnp.float32) + b_ref[...]
    t1_ref[...] = jnp.dot(af, w1_ref[...], preferred_element_type=jnp.float32)


def _tables(af, w0, w1, bias):
    return pl.pallas_call(
        _tables_body,
        grid=(N_NODES // _R_TAB,),
        in_specs=[
            pl.BlockSpec((_R_TAB, D), lambda i: (i, 0)),
            pl.BlockSpec((D, SG), lambda i: (0, 0)),
            pl.BlockSpec((D, SG), lambda i: (0, 0)),
            pl.BlockSpec((1, SG), lambda i: (0, 0)),
        ],
        out_specs=[
            pl.BlockSpec((_R_TAB, SG), lambda i: (i, 0)),
            pl.BlockSpec((_R_TAB, SG), lambda i: (i, 0)),
        ],
        out_shape=[
            jax.ShapeDtypeStruct((N_NODES, SG), jnp.float32),
            jax.ShapeDtypeStruct((N_NODES, SG), jnp.float32),
        ],
    )(af, w0, w1, bias)


def _update_tables_body(af_ref, p0_ref, p1_ref, w0_ref, w1_ref, b_ref,
                        afn_ref, t0_ref, t1_ref):
    afn = _softplus(af_ref[...] + p0_ref[...] + p1_ref[...])
    afn_ref[...] = afn
    t0_ref[...] = jnp.dot(afn, w0_ref[...], preferred_element_type=jnp.float32) + b_ref[...]
    t1_ref[...] = jnp.dot(afn, w1_ref[...], preferred_element_type=jnp.float32)


def _update_tables(af, p0, p1, w0, w1, bias):
    return pl.pallas_call(
        _update_tables_body,
        grid=(N_NODES // _R_TAB,),
        in_specs=[
            pl.BlockSpec((_R_TAB, D), lambda i: (i, 0)),
            pl.BlockSpec((_R_TAB, D), lambda i: (i, 0)),
            pl.BlockSpec((_R_TAB, D), lambda i: (i, 0)),
            pl.BlockSpec((D, SG), lambda i: (0, 0)),
            pl.BlockSpec((D, SG), lambda i: (0, 0)),
            pl.BlockSpec((1, SG), lambda i: (0, 0)),
        ],
        out_specs=[
            pl.BlockSpec((_R_TAB, D), lambda i: (i, 0)),
            pl.BlockSpec((_R_TAB, SG), lambda i: (i, 0)),
            pl.BlockSpec((_R_TAB, SG), lambda i: (i, 0)),
        ],
        out_shape=[
            jax.ShapeDtypeStruct((N_NODES, D), jnp.float32),
            jax.ShapeDtypeStruct((N_NODES, SG), jnp.float32),
            jax.ShapeDtypeStruct((N_NODES, SG), jnp.float32),
        ],
    )(af, p0, p1, w0, w1, bias)


def _final_update_body(af_ref, p0_ref, p1_ref, out_ref):
    out_ref[...] = _softplus(af_ref[...] + p0_ref[...] + p1_ref[...])


def _final_update(af, p0, p1):
    return pl.pallas_call(
        _final_update_body,
        grid=(N_NODES // _R_TAB,),
        in_specs=[
            pl.BlockSpec((_R_TAB, D), lambda i: (i, 0)),
            pl.BlockSpec((_R_TAB, D), lambda i: (i, 0)),
            pl.BlockSpec((_R_TAB, D), lambda i: (i, 0)),
        ],
        out_specs=pl.BlockSpec((_R_TAB, D), lambda i: (i, 0)),
        out_shape=jax.ShapeDtypeStruct((N_NODES, D), jnp.float32),
    )(af, p0, p1)


def _edge_gate_body(sg_ref, e_ref, we_ref, out_ref):
    sg = sg_ref[...] + jnp.dot(e_ref[...], we_ref[...], preferred_element_type=jnp.float32)
    s = sg[:, :D]
    g = sg[:, D:]
    out_ref[...] = _sigmoid(s) * _softplus(g)


def _edge_gate(sg, edges, we):
    return pl.pallas_call(
        _edge_gate_body,
        grid=(N_EDGES // _R_EDGE,),
        in_specs=[
            pl.BlockSpec((_R_EDGE, SG), lambda i: (i, 0)),
            pl.BlockSpec((_R_EDGE, EDGE_DIM), lambda i: (i, 0)),
            pl.BlockSpec((EDGE_DIM, SG), lambda i: (0, 0)),
        ],
        out_specs=pl.BlockSpec((_R_EDGE, D), lambda i: (i, 0)),
        out_shape=jax.ShapeDtypeStruct((N_EDGES, D), jnp.float32),
    )(sg, edges, we)


_SC_MESH = plsc.VectorSubcoreMesh(core_axis_name="c", subcore_axis_name="s")


@functools.partial(
    pl.kernel,
    out_type=jax.ShapeDtypeStruct((N_EDGES, SG), jnp.float32),
    mesh=_SC_MESH,
    scratch_types=[
        pltpu.VMEM((EW,), jnp.int32),
        pltpu.VMEM((EW,), jnp.int32),
        pltpu.VMEM((CHUNK, SG), jnp.float32),
        pltpu.VMEM((CHUNK, SG), jnp.float32),
        pltpu.VMEM((CHUNK, SG), jnp.float32),
        pltpu.VMEM((CHUNK, SG), jnp.float32),
        pltpu.SemaphoreType.DMA,
        pltpu.SemaphoreType.DMA,
        pltpu.SemaphoreType.DMA,
        pltpu.SemaphoreType.DMA,
    ],
)
def _sc_gather(t0_hbm, t1_hbm, i0_hbm, i1_hbm, out_hbm,
               i0all, i1all, bufa0, bufb0, bufa1, bufb1,
               gsem0, gsem1, ssem0, ssem1):
    wid = lax.axis_index("s") * NC + lax.axis_index("c")
    base = wid * EW

    pltpu.sync_copy(i0_hbm.at[pl.ds(base, EW)], i0all)
    pltpu.sync_copy(i1_hbm.at[pl.ds(base, EW)], i1all)

    def start(j, bufa, bufb, gsem):
        o = j * CHUNK
        pltpu.async_copy(t0_hbm.at[i0all.at[pl.ds(o, CHUNK)]], bufa, gsem)
        pltpu.async_copy(t1_hbm.at[i1all.at[pl.ds(o, CHUNK)]], bufb, gsem)

    def wait_g(bufa, bufb, gsem):
        pltpu.make_async_copy(t0_hbm.at[i0all.at[pl.ds(0, CHUNK)]], bufa, gsem).wait()
        pltpu.make_async_copy(t1_hbm.at[i1all.at[pl.ds(0, CHUNK)]], bufb, gsem).wait()

    def add(bufa, bufb):
        def row(r, c2):
            for c in range(SG // 16):
                sl = pl.ds(c * 16, 16)
                bufa[r, sl] = bufa[r, sl] + bufb[r, sl]
            return c2
        lax.fori_loop(0, CHUNK, row, 0)

    start(0, bufa0, bufb0, gsem0)
    start(1, bufa1, bufb1, gsem1)

    def pair(i, carry):
        j0 = 2 * i
        j1 = 2 * i + 1
        wait_g(bufa0, bufb0, gsem0)
        add(bufa0, bufb0)
        pltpu.async_copy(bufa0, out_hbm.at[pl.ds(base + j0 * CHUNK, CHUNK)], ssem0)
        wait_g(bufa1, bufb1, gsem1)
        add(bufa1, bufb1)
        pltpu.async_copy(bufa1, out_hbm.at[pl.ds(base + j1 * CHUNK, CHUNK)], ssem1)
        pltpu.make_async_copy(bufa0, out_hbm.at[pl.ds(base, CHUNK)], ssem0).wait()

        @pl.when(j0 + 2 < N_CHUNKS)
        def _():
            start(j0 + 2, bufa0, bufb0, gsem0)

        pltpu.make_async_copy(bufa1, out_hbm.at[pl.ds(base, CHUNK)], ssem1).wait()

        @pl.when(j1 + 2 < N_CHUNKS)
        def _():
            start(j1 + 2, bufa1, bufb1, gsem1)

        return carry

    lax.fori_loop(0, N_CHUNKS // 2, pair, 0)


@functools.partial(
    pl.kernel,
    out_type=jax.ShapeDtypeStruct((NC, N_NODES, D), jnp.float32),
    mesh=_SC_MESH,
    scratch_types=[
        pltpu.VMEM((CHUNK,), jnp.int32),
        pltpu.VMEM((CHUNK, D), jnp.float32),
        pltpu.VMEM((CHUNK,), jnp.int32),
        pltpu.VMEM((CHUNK, D), jnp.float32),
        pltpu.VMEM_SHARED((N_NODES, D), jnp.float32),
        pltpu.SemaphoreType.DMA,
        pltpu.SemaphoreType.DMA,
    ],
)
def _sc_segsum(t_hbm, i0_hbm, z_hbm, out_hbm, i0v0, tv0, i0v1, tv1,
               agg_sh, lsem0, lsem1):
    cid = lax.axis_index("c")
    sid = lax.axis_index("s")
    wid = sid * NC + cid
    base = wid * EW

    pltpu.sync_copy(z_hbm.at[pl.ds(sid * ROWS_PER_TILE, ROWS_PER_TILE)],
                    agg_sh.at[pl.ds(sid * ROWS_PER_TILE, ROWS_PER_TILE)])

    @pl.when(sid == NS - 1)
    def _():
        pltpu.sync_copy(z_hbm.at[pl.ds(TAIL_ROW0, TAIL_ROWS)],
                        agg_sh.at[pl.ds(TAIL_ROW0, TAIL_ROWS)])

    plsc.subcore_barrier()

    def start(j, i0v, tv, lsem):
        off = base + j * CHUNK
        pltpu.async_copy(i0_hbm.at[pl.ds(off, CHUNK)], i0v, lsem)
        pltpu.async_copy(t_hbm.at[pl.ds(off, CHUNK)], tv, lsem)

    def wait_l(i0v, tv, lsem):
        pltpu.make_async_copy(i0_hbm.at[pl.ds(base, CHUNK)], i0v, lsem).wait()
        pltpu.make_async_copy(t_hbm.at[pl.ds(base, CHUNK)], tv, lsem).wait()

    start(0, i0v0, tv0, lsem0)
    start(1, i0v1, tv1, lsem1)

    def pair(i, carry):
        j0 = 2 * i
        j1 = 2 * i + 1
        wait_l(i0v0, tv0, lsem0)
        pltpu.sync_copy(tv0, agg_sh.at[i0v0], add=True)

        @pl.when(j0 + 2 < N_CHUNKS)
        def _():
            start(j0 + 2, i0v0, tv0, lsem0)

        wait_l(i0v1, tv1, lsem1)
        pltpu.sync_copy(tv1, agg_sh.at[i0v1], add=True)

        @pl.when(j1 + 2 < N_CHUNKS)
        def _():
            start(j1 + 2, i0v1, tv1, lsem1)

        return carry

    lax.fori_loop(0, N_CHUNKS // 2, pair, 0)
    plsc.subcore_barrier()
    pltpu.sync_copy(agg_sh.at[pl.ds(sid * ROWS_PER_TILE, ROWS_PER_TILE)],
                    out_hbm.at[cid, pl.ds(sid * ROWS_PER_TILE, ROWS_PER_TILE)])

    @pl.when(sid == NS - 1)
    def _():
        pltpu.sync_copy(agg_sh.at[pl.ds(TAIL_ROW0, TAIL_ROWS)],
                        out_hbm.at[cid, pl.ds(TAIL_ROW0, TAIL_ROWS)])


def kernel(atom_features, edges_sph_features, state_attrs, pair_indices,
           atom_graph_indices, bond_graph_indices, kernel_s, bias_s,
           kernel_g, bias_g):
    del state_attrs, atom_graph_indices, bond_graph_indices
    idx = pair_indices.astype(jnp.int32)
    i0 = idx[:, 0]
    i1 = idx[:, 1]
    w0 = jnp.concatenate([kernel_s[:D], kernel_g[:D]], axis=1)
    w1 = jnp.concatenate([kernel_s[D:2 * D], kernel_g[D:2 * D]], axis=1)
    we = jnp.concatenate([kernel_s[2 * D:], kernel_g[2 * D:]], axis=1)
    bias = jnp.concatenate([bias_s, bias_g]).reshape(1, SG)
    zeros = jnp.zeros((N_NODES, D), jnp.float32)

    af = atom_features
    t0, t1 = _tables(af, w0, w1, bias)

    sg = _sc_gather(t0, t1, i0, i1)
    t = _edge_gate(sg, edges_sph_features, we)
    p = _sc_segsum(t, i0, zeros)

    af, t0, t1 = _update_tables(af, p[0], p[1], w0, w1, bias)

    sg = _sc_gather(t0, t1, i0, i1)
    t = _edge_gate(sg, edges_sph_features, we)
    p = _sc_segsum(t, i0, zeros)

    return _final_update(af, p[0], p[1])


# speedup vs baseline: 5.5812x; 1.8335x over previous
import functools

import jax
import jax.numpy as jnp
from jax import lax
from jax.experimental import pallas as pl
from jax.experimental.pallas import tpu as pltpu
from jax.experimental.pallas import tpu_sc as plsc

N_NODES = 10000
N_EDGES = 320000
D = 128
EDGE_DIM = 16
SG = 2 * D

NC = 2
NS = 16
NW = NC * NS
EW = N_EDGES // NW
CHUNK = 40
N_CHUNKS = EW // CHUNK
ROWS_PER_TILE = 624
TAIL_ROW0 = ROWS_PER_TILE * NS
TAIL_ROWS = N_NODES - TAIL_ROW0


def _softplus(x):
    return jnp.maximum(x, 0.0) + jnp.log(1.0 + jnp.exp(-jnp.abs(x)))


def _sigmoid(x):
    return 1.0 / (1.0 + jnp.exp(-x))


_R_TAB = 1000
_R_EDGE = 2000


def _tables_body(af_ref, w0_ref, w1_ref, b_ref, t0_ref, t1_ref):
    af = af_ref[...]
    t0_ref[...] = jnp.dot(af, w0_ref[...], preferred_element_type=jnp.float32) + b_ref[...]
    t1_ref[...] = jnp.dot(af, w1_ref[...], preferred_element_type=jnp.float32)


def _tables(af, w0, w1, bias):
    return pl.pallas_call(
        _tables_body,
        grid=(N_NODES // _R_TAB,),
        in_specs=[
            pl.BlockSpec((_R_TAB, D), lambda i: (i, 0)),
            pl.BlockSpec((D, SG), lambda i: (0, 0)),
            pl.BlockSpec((D, SG), lambda i: (0, 0)),
            pl.BlockSpec((1, SG), lambda i: (0, 0)),
        ],
        out_specs=[
            pl.BlockSpec((_R_TAB, SG), lambda i: (i, 0)),
            pl.BlockSpec((_R_TAB, SG), lambda i: (i, 0)),
        ],
        out_shape=[
            jax.ShapeDtypeStruct((N_NODES, SG), jnp.float32),
            jax.ShapeDtypeStruct((N_NODES, SG), jnp.float32),
        ],
    )(af, w0, w1, bias)


def _update_tables_body(af_ref, p0_ref, p1_ref, w0_ref, w1_ref, b_ref,
                        afn_ref, t0_ref, t1_ref):
    afn = _softplus(af_ref[...] + p0_ref[...] + p1_ref[...])
    afn_ref[...] = afn
    t0_ref[...] = jnp.dot(afn, w0_ref[...], preferred_element_type=jnp.float32) + b_ref[...]
    t1_ref[...] = jnp.dot(afn, w1_ref[...], preferred_element_type=jnp.float32)


def _update_tables(af, p0, p1, w0, w1, bias):
    return pl.pallas_call(
        _update_tables_body,
        grid=(N_NODES // _R_TAB,),
        in_specs=[
            pl.BlockSpec((_R_TAB, D), lambda i: (i, 0)),
            pl.BlockSpec((_R_TAB, D), lambda i: (i, 0)),
            pl.BlockSpec((_R_TAB, D), lambda i: (i, 0)),
            pl.BlockSpec((D, SG), lambda i: (0, 0)),
            pl.BlockSpec((D, SG), lambda i: (0, 0)),
            pl.BlockSpec((1, SG), lambda i: (0, 0)),
        ],
        out_specs=[
            pl.BlockSpec((_R_TAB, D), lambda i: (i, 0)),
            pl.BlockSpec((_R_TAB, SG), lambda i: (i, 0)),
            pl.BlockSpec((_R_TAB, SG), lambda i: (i, 0)),
        ],
        out_shape=[
            jax.ShapeDtypeStruct((N_NODES, D), jnp.float32),
            jax.ShapeDtypeStruct((N_NODES, SG), jnp.float32),
            jax.ShapeDtypeStruct((N_NODES, SG), jnp.float32),
        ],
    )(af, p0, p1, w0, w1, bias)


def _final_update_body(af_ref, p0_ref, p1_ref, out_ref):
    out_ref[...] = _softplus(af_ref[...] + p0_ref[...] + p1_ref[...])


def _final_update(af, p0, p1):
    return pl.pallas_call(
        _final_update_body,
        grid=(N_NODES // _R_TAB,),
        in_specs=[
            pl.BlockSpec((_R_TAB, D), lambda i: (i, 0)),
            pl.BlockSpec((_R_TAB, D), lambda i: (i, 0)),
            pl.BlockSpec((_R_TAB, D), lambda i: (i, 0)),
        ],
        out_specs=pl.BlockSpec((_R_TAB, D), lambda i: (i, 0)),
        out_shape=jax.ShapeDtypeStruct((N_NODES, D), jnp.float32),
    )(af, p0, p1)


def _edge_gate_body(sg_ref, e_ref, we_ref, out_ref):
    sg = sg_ref[...] + jnp.dot(e_ref[...], we_ref[...], preferred_element_type=jnp.float32)
    s = sg[:, :D]
    g = sg[:, D:]
    out_ref[...] = _sigmoid(s) * _softplus(g)


def _edge_gate(sg, edges, we):
    return pl.pallas_call(
        _edge_gate_body,
        grid=(N_EDGES // _R_EDGE,),
        in_specs=[
            pl.BlockSpec((_R_EDGE, SG), lambda i: (i, 0)),
            pl.BlockSpec((_R_EDGE, EDGE_DIM), lambda i: (i, 0)),
            pl.BlockSpec((EDGE_DIM, SG), lambda i: (0, 0)),
        ],
        out_specs=pl.BlockSpec((_R_EDGE, D), lambda i: (i, 0)),
        out_shape=jax.ShapeDtypeStruct((N_EDGES, D), jnp.float32),
    )(sg, edges, we)


_SC_MESH = plsc.VectorSubcoreMesh(core_axis_name="c", subcore_axis_name="s")


GCHUNK = 80
SUPER = 400
N_SUPER = EW // SUPER
NSUB = SUPER // GCHUNK
WMAX = 32


@functools.partial(
    pl.kernel,
    out_type=jax.ShapeDtypeStruct((N_EDGES, SG), jnp.float32),
    mesh=_SC_MESH,
    scratch_types=[
        pltpu.VMEM((SUPER + 16,), jnp.int32),
        pltpu.VMEM((SUPER + 16,), jnp.int32),
        pltpu.VMEM((SUPER + 16,), jnp.int32),
        pltpu.VMEM((SUPER + 16,), jnp.int32),
        pltpu.VMEM((WMAX, SG), jnp.float32),
        pltpu.VMEM((WMAX, SG), jnp.float32),
        pltpu.VMEM((WMAX, SG), jnp.float32),
        pltpu.VMEM((WMAX, SG), jnp.float32),
        pltpu.VMEM((GCHUNK, SG), jnp.float32),
        pltpu.VMEM((GCHUNK, SG), jnp.float32),
        pltpu.VMEM((GCHUNK,), jnp.int32),
        pltpu.VMEM((GCHUNK,), jnp.int32),
        pltpu.SemaphoreType.DMA,
        pltpu.SemaphoreType.DMA,
        pltpu.SemaphoreType.DMA,
        pltpu.SemaphoreType.DMA,
        pltpu.SemaphoreType.DMA,
    ],
)
def _sc_gather(t0_hbm, t1_hbm, i0_hbm, i1_hbm, out_hbm,
               i0sA, i1sA, i0sB, i1sB,
               w0A, w1A, w0B, w1B, bufo0, bufo1, idx0v, idx1v,
               wsemA, wsemB, ssem0, ssem1, fsem):
    wid = lax.axis_index("s") * NC + lax.axis_index("c")
    base = wid * EW
    bufos = (bufo0, bufo1)
    ssems = (ssem0, ssem1)

    def issue(k, i0s, i1s, w0, w1, wsem):
        off = base + k * SUPER
        pltpu.sync_copy(i0_hbm.at[pl.ds(off, SUPER)], i0s.at[pl.ds(0, SUPER)])
        pltpu.sync_copy(i1_hbm.at[pl.ds(off, SUPER)], i1s.at[pl.ds(0, SUPER)])
        head0 = i0s[pl.ds(0, 16)]
        tail0 = i0s[pl.ds(SUPER - 16, 16)]
        head1 = i1s[pl.ds(0, 16)]
        tail1 = i1s[pl.ds(SUPER - 16, 16)]
        lo0, hi0 = head0[0], tail0[15]
        lo1, hi1 = head1[0], tail1[15]
        st0 = jnp.minimum((lo0 // 8) * 8, N_NODES - WMAX)
        st1 = jnp.minimum((lo1 // 8) * 8, N_NODES - WMAX)
        ok = jnp.logical_and(hi0 - st0 < WMAX, hi1 - st1 < WMAX)

        @pl.when(ok)
        def _():
            pltpu.async_copy(t0_hbm.at[pl.ds(pl.multiple_of(st0, 8), WMAX)], w0, wsem)
            pltpu.async_copy(t1_hbm.at[pl.ds(pl.multiple_of(st1, 8), WMAX)], w1, wsem)

        return ok.astype(jnp.int32), st0, st1

    def store(sub, off, p):
        pltpu.async_copy(bufos[p], out_hbm.at[pl.ds(off + sub * GCHUNK, GCHUNK)], ssems[p])

    def wait_store(p):
        pltpu.make_async_copy(bufos[p], out_hbm.at[pl.ds(base, GCHUNK)], ssems[p]).wait()

    def process(k, i0s, i1s, w0, w1, wsem, state):
        ok, st0, st1 = state
        off = base + k * SUPER

        def fast():
            pltpu.make_async_copy(t0_hbm.at[pl.ds(0, WMAX)], w0, wsem).wait()
            pltpu.make_async_copy(t1_hbm.at[pl.ds(0, WMAX)], w1, wsem).wait()
            for sub in range(NSUB):
                p = sub % 2
                if sub >= 2:
                    wait_store(p)
                bo = bufos[p]

                def grp(g, c2, _sub=sub, _bo=bo):
                    e0 = _sub * GCHUNK + g * 8
                    r0v = i0s[pl.ds(e0, 16)] - st0
                    r1v = i1s[pl.ds(e0, 16)] - st1
                    for lane in range(8):
                        r0 = r0v[lane]
                        r1 = r1v[lane]
                        el = g * 8 + lane
                        va = [w0[r0, pl.ds(c * 16, 16)] for c in range(SG // 16)]
                        vb = [w1[r1, pl.ds(c * 16, 16)] for c in range(SG // 16)]
                        for c in range(SG // 16):
                            _bo[el, pl.ds(c * 16, 16)] = va[c] + vb[c]
                    return c2

                lax.fori_loop(0, GCHUNK // 8, grp, 0)
                store(sub, off, p)
            wait_store(NSUB % 2)
            wait_store((NSUB - 1) % 2)

        def slow():
            def sub_body(sub, c3):
                o = off + sub * GCHUNK
                pltpu.sync_copy(i0_hbm.at[pl.ds(o, GCHUNK)], idx0v)
                pltpu.sync_copy(i1_hbm.at[pl.ds(o, GCHUNK)], idx1v)
                cpa = pltpu.async_copy(t0_hbm.at[idx0v], bufo0, fsem)
                cpb = pltpu.async_copy(t1_hbm.at[idx1v], bufo1, fsem)
                cpa.wait()
                cpb.wait()

                def row(r, c2):
                    for c in range(SG // 16):
                        sl = pl.ds(c * 16, 16)
                        bufo0[r, sl] = bufo0[r, sl] + bufo1[r, sl]
                    return c2

                lax.fori_loop(0, GCHUNK, row, 0)
                pltpu.sync_copy(bufo0, out_hbm.at[pl.ds(o, GCHUNK)])
                return c3

            lax.fori_loop(0, NSUB, sub_body, 0)

        lax.cond(ok == 1, fast, slow)

    stateA = issue(0, i0sA, i1sA, w0A, w1A, wsemA)

    def pair(i, stA):
        k0 = 2 * i
        k1 = 2 * i + 1
        stB = issue(k1, i0sB, i1sB, w0B, w1B, wsemB)
        process(k0, i0sA, i1sA, w0A, w1A, wsemA, stA)
        stA_next = lax.cond(
            k0 + 2 < N_SUPER,
            lambda: issue(k0 + 2, i0sA, i1sA, w0A, w1A, wsemA),
            lambda: (jnp.int32(0), jnp.int32(0), jnp.int32(0)),
        )
        process(k1, i0sB, i1sB, w0B, w1B, wsemB, stB)
        return stA_next

    st_last = lax.fori_loop(0, N_SUPER // 2, pair, stateA)
    process(N_SUPER - 1, i0sA, i1sA, w0A, w1A, wsemA, st_last)


@functools.partial(
    pl.kernel,
    out_type=jax.ShapeDtypeStruct((NC, N_NODES, D), jnp.float32),
    mesh=_SC_MESH,
    scratch_types=[
        pltpu.VMEM((CHUNK,), jnp.int32),
        pltpu.VMEM((CHUNK, D), jnp.float32),
        pltpu.VMEM((CHUNK,), jnp.int32),
        pltpu.VMEM((CHUNK, D), jnp.float32),
        pltpu.VMEM_SHARED((N_NODES, D), jnp.float32),
        pltpu.SemaphoreType.DMA,
        pltpu.SemaphoreType.DMA,
    ],
)
def _sc_segsum(t_hbm, i0_hbm, z_hbm, out_hbm, i0v0, tv0, i0v1, tv1,
               agg_sh, lsem0, lsem1):
    cid = lax.axis_index("c")
    sid = lax.axis_index("s")
    wid = sid * NC + cid
    base = wid * EW

    pltpu.sync_copy(z_hbm.at[pl.ds(sid * ROWS_PER_TILE, ROWS_PER_TILE)],
                    agg_sh.at[pl.ds(sid * ROWS_PER_TILE, ROWS_PER_TILE)])

    @pl.when(sid == NS - 1)
    def _():
        pltpu.sync_copy(z_hbm.at[pl.ds(TAIL_ROW0, TAIL_ROWS)],
                        agg_sh.at[pl.ds(TAIL_ROW0, TAIL_ROWS)])

    plsc.subcore_barrier()

    def start(j, i0v, tv, lsem):
        off = base + j * CHUNK
        pltpu.async_copy(i0_hbm.at[pl.ds(off, CHUNK)], i0v, lsem)
        pltpu.async_copy(t_hbm.at[pl.ds(off, CHUNK)], tv, lsem)

    def wait_l(i0v, tv, lsem):
        pltpu.make_async_copy(i0_hbm.at[pl.ds(base, CHUNK)], i0v, lsem).wait()
        pltpu.make_async_copy(t_hbm.at[pl.ds(base, CHUNK)], tv, lsem).wait()

    start(0, i0v0, tv0, lsem0)
    start(1, i0v1, tv1, lsem1)

    def pair(i, carry):
        j0 = 2 * i
        j1 = 2 * i + 1
        wait_l(i0v0, tv0, lsem0)
        pltpu.sync_copy(tv0, agg_sh.at[i0v0], add=True)

        @pl.when(j0 + 2 < N_CHUNKS)
        def _():
            start(j0 + 2, i0v0, tv0, lsem0)

        wait_l(i0v1, tv1, lsem1)
        pltpu.sync_copy(tv1, agg_sh.at[i0v1], add=True)

        @pl.when(j1 + 2 < N_CHUNKS)
        def _():
            start(j1 + 2, i0v1, tv1, lsem1)

        return carry

    lax.fori_loop(0, N_CHUNKS // 2, pair, 0)
    plsc.subcore_barrier()
    pltpu.sync_copy(agg_sh.at[pl.ds(sid * ROWS_PER_TILE, ROWS_PER_TILE)],
                    out_hbm.at[cid, pl.ds(sid * ROWS_PER_TILE, ROWS_PER_TILE)])

    @pl.when(sid == NS - 1)
    def _():
        pltpu.sync_copy(agg_sh.at[pl.ds(TAIL_ROW0, TAIL_ROWS)],
                        out_hbm.at[cid, pl.ds(TAIL_ROW0, TAIL_ROWS)])


def kernel(atom_features, edges_sph_features, state_attrs, pair_indices,
           atom_graph_indices, bond_graph_indices, kernel_s, bias_s,
           kernel_g, bias_g):
    del state_attrs, atom_graph_indices, bond_graph_indices
    idx = pair_indices.astype(jnp.int32)
    i0 = idx[:, 0]
    i1 = idx[:, 1]
    w0 = jnp.concatenate([kernel_s[:D], kernel_g[:D]], axis=1)
    w1 = jnp.concatenate([kernel_s[D:2 * D], kernel_g[D:2 * D]], axis=1)
    we = jnp.concatenate([kernel_s[2 * D:], kernel_g[2 * D:]], axis=1)
    bias = jnp.concatenate([bias_s, bias_g]).reshape(1, SG)
    zeros = jnp.zeros((N_NODES, D), jnp.float32)

    af = atom_features
    t0, t1 = _tables(af, w0, w1, bias)

    sg = _sc_gather(t0, t1, i0, i1)
    t = _edge_gate(sg, edges_sph_features, we)
    p = _sc_segsum(t, i0, zeros)

    af, t0, t1 = _update_tables(af, p[0], p[1], w0, w1, bias)

    sg = _sc_gather(t0, t1, i0, i1)
    t = _edge_gate(sg, edges_sph_features, we)
    p = _sc_segsum(t, i0, zeros)

    return _final_update(af, p[0], p[1])


# speedup vs baseline: 6.4543x; 1.1564x over previous
import functools

import jax
import jax.numpy as jnp
from jax import lax
from jax.experimental import pallas as pl
from jax.experimental.pallas import tpu as pltpu
from jax.experimental.pallas import tpu_sc as plsc

N_NODES = 10000
N_EDGES = 320000
D = 128
EDGE_DIM = 16
SG = 2 * D

NC = 2
NS = 16
NW = NC * NS
ROWS_PER_TILE = 624
TAIL_ROW0 = ROWS_PER_TILE * NS
TAIL_ROWS = N_NODES - TAIL_ROW0

EH = N_EDGES // 2


def _softplus(x):
    return jnp.maximum(x, 0.0) + jnp.log(1.0 + jnp.exp(-jnp.abs(x)))


def _sigmoid(x):
    return 1.0 / (1.0 + jnp.exp(-x))


_R_TAB = 1000
_R_EDGE = 2000


def _tables_body(af_ref, w0_ref, w1_ref, b_ref, t0_ref, t1_ref):
    af = af_ref[...]
    t0_ref[...] = jnp.dot(af, w0_ref[...], preferred_element_type=jnp.float32) + b_ref[...]
    t1_ref[...] = jnp.dot(af, w1_ref[...], preferred_element_type=jnp.float32)


def _tables(af, w0, w1, bias):
    return pl.pallas_call(
        _tables_body,
        grid=(N_NODES // _R_TAB,),
        in_specs=[
            pl.BlockSpec((_R_TAB, D), lambda i: (i, 0)),
            pl.BlockSpec((D, SG), lambda i: (0, 0)),
            pl.BlockSpec((D, SG), lambda i: (0, 0)),
            pl.BlockSpec((1, SG), lambda i: (0, 0)),
        ],
        out_specs=[
            pl.BlockSpec((_R_TAB, SG), lambda i: (i, 0)),
            pl.BlockSpec((_R_TAB, SG), lambda i: (i, 0)),
        ],
        out_shape=[
            jax.ShapeDtypeStruct((N_NODES, SG), jnp.float32),
            jax.ShapeDtypeStruct((N_NODES, SG), jnp.float32),
        ],
    )(af, w0, w1, bias)


def _update_tables_body(af_ref, pa_ref, pb_ref, w0_ref, w1_ref, b_ref,
                        afn_ref, t0_ref, t1_ref):
    p = (pa_ref[0] + pa_ref[1]) + (pb_ref[0] + pb_ref[1])
    afn = _softplus(af_ref[...] + p)
    afn_ref[...] = afn
    t0_ref[...] = jnp.dot(afn, w0_ref[...], preferred_element_type=jnp.float32) + b_ref[...]
    t1_ref[...] = jnp.dot(afn, w1_ref[...], preferred_element_type=jnp.float32)


def _update_tables(af, pa, pb, w0, w1, bias):
    return pl.pallas_call(
        _update_tables_body,
        grid=(N_NODES // _R_TAB,),
        in_specs=[
            pl.BlockSpec((_R_TAB, D), lambda i: (i, 0)),
            pl.BlockSpec((NC, _R_TAB, D), lambda i: (0, i, 0)),
            pl.BlockSpec((NC, _R_TAB, D), lambda i: (0, i, 0)),
            pl.BlockSpec((D, SG), lambda i: (0, 0)),
            pl.BlockSpec((D, SG), lambda i: (0, 0)),
            pl.BlockSpec((1, SG), lambda i: (0, 0)),
        ],
        out_specs=[
            pl.BlockSpec((_R_TAB, D), lambda i: (i, 0)),
            pl.BlockSpec((_R_TAB, SG), lambda i: (i, 0)),
            pl.BlockSpec((_R_TAB, SG), lambda i: (i, 0)),
        ],
        out_shape=[
            jax.ShapeDtypeStruct((N_NODES, D), jnp.float32),
            jax.ShapeDtypeStruct((N_NODES, SG), jnp.float32),
            jax.ShapeDtypeStruct((N_NODES, SG), jnp.float32),
        ],
    )(af, pa, pb, w0, w1, bias)


def _final_update_body(af_ref, pa_ref, pb_ref, out_ref):
    p = (pa_ref[0] + pa_ref[1]) + (pb_ref[0] + pb_ref[1])
    out_ref[...] = _softplus(af_ref[...] + p)


def _final_update(af, pa, pb):
    return pl.pallas_call(
        _final_update_body,
        grid=(N_NODES // _R_TAB,),
        in_specs=[
            pl.BlockSpec((_R_TAB, D), lambda i: (i, 0)),
            pl.BlockSpec((NC, _R_TAB, D), lambda i: (0, i, 0)),
            pl.BlockSpec((NC, _R_TAB, D), lambda i: (0, i, 0)),
        ],
        out_specs=pl.BlockSpec((_R_TAB, D), lambda i: (i, 0)),
        out_shape=jax.ShapeDtypeStruct((N_NODES, D), jnp.float32),
    )(af, pa, pb)


def _edge_gate_body(sg_ref, e_ref, we_ref, out_ref):
    sg = sg_ref[...] + jnp.dot(e_ref[...], we_ref[...], preferred_element_type=jnp.float32)
    s = sg[:, :D]
    g = sg[:, D:]
    out_ref[...] = _sigmoid(s) * _softplus(g)


def _edge_gate(sg, edges, we):
    n_edges = sg.shape[0]
    return pl.pallas_call(
        _edge_gate_body,
        grid=(n_edges // _R_EDGE,),
        in_specs=[
            pl.BlockSpec((_R_EDGE, SG), lambda i: (i, 0)),
            pl.BlockSpec((_R_EDGE, EDGE_DIM), lambda i: (i, 0)),
            pl.BlockSpec((EDGE_DIM, SG), lambda i: (0, 0)),
        ],
        out_specs=pl.BlockSpec((_R_EDGE, D), lambda i: (i, 0)),
        out_shape=jax.ShapeDtypeStruct((n_edges, D), jnp.float32),
    )(sg, edges, we)


_SC_MESH = plsc.VectorSubcoreMesh(core_axis_name="c", subcore_axis_name="s")


def _make_sc_gather(n_edges, super_, gchunk, wmax):
    ew = n_edges // NW
    n_super = ew // super_
    nsub = super_ // gchunk
    assert ew % super_ == 0 and super_ % gchunk == 0 and gchunk % 8 == 0
    assert n_super % 2 == 1 and wmax % 8 == 0

    @functools.partial(
        pl.kernel,
        out_type=jax.ShapeDtypeStruct((n_edges, SG), jnp.float32),
        mesh=_SC_MESH,
        scratch_types=[
            pltpu.VMEM((super_ + 16,), jnp.int32),
            pltpu.VMEM((super_ + 16,), jnp.int32),
            pltpu.VMEM((super_ + 16,), jnp.int32),
            pltpu.VMEM((super_ + 16,), jnp.int32),
            pltpu.VMEM((wmax, SG), jnp.float32),
            pltpu.VMEM((wmax, SG), jnp.float32),
            pltpu.VMEM((wmax, SG), jnp.float32),
            pltpu.VMEM((wmax, SG), jnp.float32),
            pltpu.VMEM((gchunk, SG), jnp.float32),
            pltpu.VMEM((gchunk, SG), jnp.float32),
            pltpu.VMEM((gchunk,), jnp.int32),
            pltpu.VMEM((gchunk,), jnp.int32),
            pltpu.SemaphoreType.DMA,
            pltpu.SemaphoreType.DMA,
            pltpu.SemaphoreType.DMA,
            pltpu.SemaphoreType.DMA,
            pltpu.SemaphoreType.DMA,
        ],
    )
    def _sc_gather(t0_hbm, t1_hbm, i0_hbm, i1_hbm, out_hbm,
                   i0sA, i1sA, i0sB, i1sB,
                   w0A, w1A, w0B, w1B, bufo0, bufo1, idx0v, idx1v,
                   wsemA, wsemB, ssem0, ssem1, fsem):
        wid = lax.axis_index("s") * NC + lax.axis_index("c")
        base = wid * ew
        bufos = (bufo0, bufo1)
        ssems = (ssem0, ssem1)

        def issue(k, i0s, i1s, w0, w1, wsem):
            off = base + k * super_
            pltpu.sync_copy(i0_hbm.at[pl.ds(off, super_)], i0s.at[pl.ds(0, super_)])
            pltpu.sync_copy(i1_hbm.at[pl.ds(off, super_)], i1s.at[pl.ds(0, super_)])
            head0 = i0s[pl.ds(0, 16)]
            tail0 = i0s[pl.ds(super_ - 16, 16)]
            head1 = i1s[pl.ds(0, 16)]
            tail1 = i1s[pl.ds(super_ - 16, 16)]
            lo0, hi0 = head0[0], tail0[15]
            lo1, hi1 = head1[0], tail1[15]
            st0 = jnp.minimum((lo0 // 8) * 8, N_NODES - wmax)
            st1 = jnp.minimum((lo1 // 8) * 8, N_NODES - wmax)
            ok = jnp.logical_and(hi0 - st0 < wmax, hi1 - st1 < wmax)

            @pl.when(ok)
            def _():
                pltpu.async_copy(t0_hbm.at[pl.ds(pl.multiple_of(st0, 8), wmax)], w0, wsem)
                pltpu.async_copy(t1_hbm.at[pl.ds(pl.multiple_of(st1, 8), wmax)], w1, wsem)

            return ok.astype(jnp.int32), st0, st1

        def store(sub, off, p):
            pltpu.async_copy(bufos[p], out_hbm.at[pl.ds(off + sub * gchunk, gchunk)], ssems[p])

        def wait_store(p):
            pltpu.make_async_copy(bufos[p], out_hbm.at[pl.ds(base, gchunk)], ssems[p]).wait()

        def process(k, i0s, i1s, w0, w1, wsem, state):
            ok, st0, st1 = state
            off = base + k * super_

            def fast():
                pltpu.make_async_copy(t0_hbm.at[pl.ds(0, wmax)], w0, wsem).wait()
                pltpu.make_async_copy(t1_hbm.at[pl.ds(0, wmax)], w1, wsem).wait()
                for sub in range(nsub):
                    p = sub % 2
                    if sub >= 2:
                        wait_store(p)
                    bo = bufos[p]

                    def grp(g, c2, _sub=sub, _bo=bo):
                        e0 = _sub * gchunk + g * 8
                        r0v = i0s[pl.ds(e0, 16)] - st0
                        r1v = i1s[pl.ds(e0, 16)] - st1
                        for lane in range(8):
                            r0 = r0v[lane]
                            r1 = r1v[lane]
                            el = g * 8 + lane
                            va = [w0[r0, pl.ds(c * 16, 16)] for c in range(SG // 16)]
                            vb = [w1[r1, pl.ds(c * 16, 16)] for c in range(SG // 16)]
                            for c in range(SG // 16):
                                _bo[el, pl.ds(c * 16, 16)] = va[c] + vb[c]
                        return c2

                    lax.fori_loop(0, gchunk // 8, grp, 0)
                    store(sub, off, p)
                wait_store(nsub % 2)
                wait_store((nsub - 1) % 2)

            def slow():
                def sub_body(sub, c3):
                    o = off + sub * gchunk
                    pltpu.sync_copy(i0_hbm.at[pl.ds(o, gchunk)], idx0v)
                    pltpu.sync_copy(i1_hbm.at[pl.ds(o, gchunk)], idx1v)
                    cpa = pltpu.async_copy(t0_hbm.at[idx0v], bufo0, fsem)
                    cpb = pltpu.async_copy(t1_hbm.at[idx1v], bufo1, fsem)
                    cpa.wait()
                    cpb.wait()

                    def row(r, c2):
                        for c in range(SG // 16):
                            sl = pl.ds(c * 16, 16)
                            bufo0[r, sl] = bufo0[r, sl] + bufo1[r, sl]
                        return c2

                    lax.fori_loop(0, gchunk, row, 0)
                    pltpu.sync_copy(bufo0, out_hbm.at[pl.ds(o, gchunk)])
                    return c3

                lax.fori_loop(0, nsub, sub_body, 0)

            lax.cond(ok == 1, fast, slow)

        stateA = issue(0, i0sA, i1sA, w0A, w1A, wsemA)

        def pair(i, stA):
            k0 = 2 * i
            k1 = 2 * i + 1
            stB = issue(k1, i0sB, i1sB, w0B, w1B, wsemB)
            process(k0, i0sA, i1sA, w0A, w1A, wsemA, stA)
            stA_next = lax.cond(
                k0 + 2 < n_super,
                lambda: issue(k0 + 2, i0sA, i1sA, w0A, w1A, wsemA),
                lambda: (jnp.int32(0), jnp.int32(0), jnp.int32(0)),
            )
            process(k1, i0sB, i1sB, w0B, w1B, wsemB, stB)
            return stA_next

        st_last = lax.fori_loop(0, n_super // 2, pair, stateA)
        process(n_super - 1, i0sA, i1sA, w0A, w1A, wsemA, st_last)

    return _sc_gather


def _make_sc_segsum(n_edges, chunk):
    ew = n_edges // NW
    n_chunks = ew // chunk
    assert ew % chunk == 0 and chunk % 8 == 0 and n_chunks >= 2

    @functools.partial(
        pl.kernel,
        out_type=jax.ShapeDtypeStruct((NC, N_NODES, D), jnp.float32),
        mesh=_SC_MESH,
        scratch_types=[
            pltpu.VMEM((chunk,), jnp.int32),
            pltpu.VMEM((chunk, D), jnp.float32),
            pltpu.VMEM((chunk,), jnp.int32),
            pltpu.VMEM((chunk, D), jnp.float32),
            pltpu.VMEM_SHARED((N_NODES, D), jnp.float32),
            pltpu.SemaphoreType.DMA,
            pltpu.SemaphoreType.DMA,
        ],
    )
    def _sc_segsum(t_hbm, i0_hbm, z_hbm, out_hbm, i0v0, tv0, i0v1, tv1,
                   agg_sh, lsem0, lsem1):
        cid = lax.axis_index("c")
        sid = lax.axis_index("s")
        wid = sid * NC + cid
        base = wid * ew

        pltpu.sync_copy(z_hbm.at[pl.ds(sid * ROWS_PER_TILE, ROWS_PER_TILE)],
                        agg_sh.at[pl.ds(sid * ROWS_PER_TILE, ROWS_PER_TILE)])

        @pl.when(sid == NS - 1)
        def _():
            pltpu.sync_copy(z_hbm.at[pl.ds(TAIL_ROW0, TAIL_ROWS)],
                            agg_sh.at[pl.ds(TAIL_ROW0, TAIL_ROWS)])

        plsc.subcore_barrier()

        def start(j, i0v, tv, lsem):
            off = base + j * chunk
            pltpu.async_copy(i0_hbm.at[pl.ds(off, chunk)], i0v, lsem)
            pltpu.async_copy(t_hbm.at[pl.ds(off, chunk)], tv, lsem)

        def wait_l(i0v, tv, lsem):
            pltpu.make_async_copy(i0_hbm.at[pl.ds(base, chunk)], i0v, lsem).wait()
            pltpu.make_async_copy(t_hbm.at[pl.ds(base, chunk)], tv, lsem).wait()

        start(0, i0v0, tv0, lsem0)
        start(1, i0v1, tv1, lsem1)

        def pair(i, carry):
            j0 = 2 * i
            j1 = 2 * i + 1
            wait_l(i0v0, tv0, lsem0)
            pltpu.sync_copy(tv0, agg_sh.at[i0v0], add=True)

            @pl.when(j0 + 2 < n_chunks)
            def _():
                start(j0 + 2, i0v0, tv0, lsem0)

            wait_l(i0v1, tv1, lsem1)
            pltpu.sync_copy(tv1, agg_sh.at[i0v1], add=True)

            @pl.when(j1 + 2 < n_chunks)
            def _():
                start(j1 + 2, i0v1, tv1, lsem1)

            return carry

        lax.fori_loop(0, n_chunks // 2, pair, 0)
        if n_chunks % 2 == 1:
            wait_l(i0v0, tv0, lsem0)
            pltpu.sync_copy(tv0, agg_sh.at[i0v0], add=True)
        plsc.subcore_barrier()
        pltpu.sync_copy(agg_sh.at[pl.ds(sid * ROWS_PER_TILE, ROWS_PER_TILE)],
                        out_hbm.at[cid, pl.ds(sid * ROWS_PER_TILE, ROWS_PER_TILE)])

        @pl.when(sid == NS - 1)
        def _():
            pltpu.sync_copy(agg_sh.at[pl.ds(TAIL_ROW0, TAIL_ROWS)],
                            out_hbm.at[cid, pl.ds(TAIL_ROW0, TAIL_ROWS)])

    return _sc_segsum


_sc_gather_half = _make_sc_gather(EH, 200, 40, 32)
_sc_segsum_half = _make_sc_segsum(EH, 40)


def kernel(atom_features, edges_sph_features, state_attrs, pair_indices,
           atom_graph_indices, bond_graph_indices, kernel_s, bias_s,
           kernel_g, bias_g):
    del state_attrs, atom_graph_indices, bond_graph_indices
    idx = pair_indices.astype(jnp.int32)
    i0a, i0b = idx[:EH, 0], idx[EH:, 0]
    i1a, i1b = idx[:EH, 1], idx[EH:, 1]
    ea, eb = edges_sph_features[:EH], edges_sph_features[EH:]
    w0 = jnp.concatenate([kernel_s[:D], kernel_g[:D]], axis=1)
    w1 = jnp.concatenate([kernel_s[D:2 * D], kernel_g[D:2 * D]], axis=1)
    we = jnp.concatenate([kernel_s[2 * D:], kernel_g[2 * D:]], axis=1)
    bias = jnp.concatenate([bias_s, bias_g]).reshape(1, SG)
    zeros = jnp.zeros((N_NODES, D), jnp.float32)

    def half_chains(t0, t1):
        sga = _sc_gather_half(t0, t1, i0a, i1a)
        ta = _edge_gate(sga, ea, we)
        sgb = _sc_gather_half(t0, t1, i0b, i1b)
        pa = _sc_segsum_half(ta, i0a, zeros)
        tb = _edge_gate(sgb, eb, we)
        pb = _sc_segsum_half(tb, i0b, zeros)
        return pa, pb

    af = atom_features
    t0, t1 = _tables(af, w0, w1, bias)
    pa, pb = half_chains(t0, t1)
    af, t0, t1 = _update_tables(af, pa, pb, w0, w1, bias)
    pa, pb = half_chains(t0, t1)
    return _final_update(af, pa, pb)


# speedup vs baseline: 7.2422x; 1.1221x over previous
import functools

import jax
import jax.numpy as jnp
from jax import lax
from jax.experimental import pallas as pl
from jax.experimental.pallas import tpu as pltpu
from jax.experimental.pallas import tpu_sc as plsc

N_NODES = 10000
N_EDGES = 320000
D = 128
EDGE_DIM = 16
SG = 2 * D

NC = 2
NS = 16
NW = NC * NS
ROWS_PER_TILE = 624
TAIL_ROW0 = ROWS_PER_TILE * NS
TAIL_ROWS = N_NODES - TAIL_ROW0

EH = N_EDGES // 2


def _softplus(x):
    return jnp.maximum(x, 0.0) + jnp.log(1.0 + jnp.exp(-jnp.abs(x)))


def _sigmoid(x):
    return 1.0 / (1.0 + jnp.exp(-x))


WSG = SG // 2


def _pack_words(sg):
    si = lax.bitcast_convert_type(sg[:, :D], jnp.int32)
    gi = lax.bitcast_convert_type(sg[:, D:], jnp.int32)

    def rne(x):
        rbit = jnp.bitwise_and(jnp.right_shift(x, 16), 1)
        return jnp.right_shift(x + jnp.int32(0x7FFF) + rbit, 16)

    w = jnp.bitwise_or(jnp.bitwise_and(rne(si), jnp.int32(0xFFFF)),
                       jnp.left_shift(rne(gi), 16))
    return lax.bitcast_convert_type(w, jnp.float32)


def _unpack_words(sgw):
    wi = lax.bitcast_convert_type(sgw, jnp.int32)
    s = lax.bitcast_convert_type(jnp.left_shift(wi, 16), jnp.float32)
    g = lax.bitcast_convert_type(jnp.bitwise_and(wi, jnp.int32(-65536)), jnp.float32)
    return s, g


_R_TAB = 1000
_R_EDGE = 2000


def _tables_body(af_ref, w0_ref, w1_ref, b_ref, t0_ref, t1_ref):
    af = af_ref[...]
    t0_ref[...] = _pack_words(
        jnp.dot(af, w0_ref[...], preferred_element_type=jnp.float32) + b_ref[...])
    t1_ref[...] = _pack_words(
        jnp.dot(af, w1_ref[...], preferred_element_type=jnp.float32))


def _tables(af, w0, w1, bias):
    return pl.pallas_call(
        _tables_body,
        grid=(N_NODES // _R_TAB,),
        in_specs=[
            pl.BlockSpec((_R_TAB, D), lambda i: (i, 0)),
            pl.BlockSpec((D, SG), lambda i: (0, 0)),
            pl.BlockSpec((D, SG), lambda i: (0, 0)),
            pl.BlockSpec((1, SG), lambda i: (0, 0)),
        ],
        out_specs=[
            pl.BlockSpec((_R_TAB, WSG), lambda i: (i, 0)),
            pl.BlockSpec((_R_TAB, WSG), lambda i: (i, 0)),
        ],
        out_shape=[
            jax.ShapeDtypeStruct((N_NODES, WSG), jnp.float32),
            jax.ShapeDtypeStruct((N_NODES, WSG), jnp.float32),
        ],
    )(af, w0, w1, bias)


def _update_tables_body(af_ref, pa_ref, pb_ref, w0_ref, w1_ref, b_ref,
                        afn_ref, t0_ref, t1_ref):
    p = (pa_ref[0] + pa_ref[1]) + (pb_ref[0] + pb_ref[1])
    afn = _softplus(af_ref[...] + p)
    afn_ref[...] = afn
    t0_ref[...] = _pack_words(
        jnp.dot(afn, w0_ref[...], preferred_element_type=jnp.float32) + b_ref[...])
    t1_ref[...] = _pack_words(
        jnp.dot(afn, w1_ref[...], preferred_element_type=jnp.float32))


def _update_tables(af, pa, pb, w0, w1, bias):
    return pl.pallas_call(
        _update_tables_body,
        grid=(N_NODES // _R_TAB,),
        in_specs=[
            pl.BlockSpec((_R_TAB, D), lambda i: (i, 0)),
            pl.BlockSpec((NC, _R_TAB, D), lambda i: (0, i, 0)),
            pl.BlockSpec((NC, _R_TAB, D), lambda i: (0, i, 0)),
            pl.BlockSpec((D, SG), lambda i: (0, 0)),
            pl.BlockSpec((D, SG), lambda i: (0, 0)),
            pl.BlockSpec((1, SG), lambda i: (0, 0)),
        ],
        out_specs=[
            pl.BlockSpec((_R_TAB, D), lambda i: (i, 0)),
            pl.BlockSpec((_R_TAB, WSG), lambda i: (i, 0)),
            pl.BlockSpec((_R_TAB, WSG), lambda i: (i, 0)),
        ],
        out_shape=[
            jax.ShapeDtypeStruct((N_NODES, D), jnp.float32),
            jax.ShapeDtypeStruct((N_NODES, WSG), jnp.float32),
            jax.ShapeDtypeStruct((N_NODES, WSG), jnp.float32),
        ],
    )(af, pa, pb, w0, w1, bias)


def _final_update_body(af_ref, pa_ref, pb_ref, out_ref):
    p = (pa_ref[0] + pa_ref[1]) + (pb_ref[0] + pb_ref[1])
    out_ref[...] = _softplus(af_ref[...] + p)


def _final_update(af, pa, pb):
    return pl.pallas_call(
        _final_update_body,
        grid=(N_NODES // _R_TAB,),
        in_specs=[
            pl.BlockSpec((_R_TAB, D), lambda i: (i, 0)),
            pl.BlockSpec((NC, _R_TAB, D), lambda i: (0, i, 0)),
            pl.BlockSpec((NC, _R_TAB, D), lambda i: (0, i, 0)),
        ],
        out_specs=pl.BlockSpec((_R_TAB, D), lambda i: (i, 0)),
        out_shape=jax.ShapeDtypeStruct((N_NODES, D), jnp.float32),
    )(af, pa, pb)


def _edge_gate_body(sgw_ref, e_ref, we_ref, out_ref):
    w = sgw_ref[...]
    s0, g0 = _unpack_words(w[:, :WSG])
    s1, g1 = _unpack_words(w[:, WSG:])
    et = jnp.dot(e_ref[...], we_ref[...], preferred_element_type=jnp.float32)
    s = (s0 + s1) + et[:, :D]
    g = (g0 + g1) + et[:, D:]
    out_ref[...] = _sigmoid(s) * _softplus(g)


def _edge_gate(sgw, edges, we):
    n_edges = sgw.shape[0]
    return pl.pallas_call(
        _edge_gate_body,
        grid=(n_edges // _R_EDGE,),
        in_specs=[
            pl.BlockSpec((_R_EDGE, SG), lambda i: (i, 0)),
            pl.BlockSpec((_R_EDGE, EDGE_DIM), lambda i: (i, 0)),
            pl.BlockSpec((EDGE_DIM, SG), lambda i: (0, 0)),
        ],
        out_specs=pl.BlockSpec((_R_EDGE, D), lambda i: (i, 0)),
        out_shape=jax.ShapeDtypeStruct((n_edges, D), jnp.float32),
    )(sgw, edges, we)


_SC_MESH = plsc.VectorSubcoreMesh(core_axis_name="c", subcore_axis_name="s")


def _make_sc_gather(n_edges, super_, gchunk, wmax):
    ew = n_edges // NW
    n_super = ew // super_
    nsub = super_ // gchunk
    assert ew % super_ == 0 and super_ % gchunk == 0 and gchunk % 8 == 0
    assert n_super % 2 == 1 and wmax % 8 == 0

    @functools.partial(
        pl.kernel,
        out_type=jax.ShapeDtypeStruct((n_edges, SG), jnp.float32),
        mesh=_SC_MESH,
        scratch_types=[
            pltpu.VMEM((super_ + 16,), jnp.int32),
            pltpu.VMEM((super_ + 16,), jnp.int32),
            pltpu.VMEM((super_ + 16,), jnp.int32),
            pltpu.VMEM((super_ + 16,), jnp.int32),
            pltpu.VMEM((wmax, WSG), jnp.float32),
            pltpu.VMEM((wmax, WSG), jnp.float32),
            pltpu.VMEM((wmax, WSG), jnp.float32),
            pltpu.VMEM((wmax, WSG), jnp.float32),
            pltpu.VMEM((gchunk, SG), jnp.float32),
            pltpu.VMEM((gchunk, SG), jnp.float32),
            pltpu.VMEM((gchunk, WSG), jnp.float32),
            pltpu.VMEM((gchunk, WSG), jnp.float32),
            pltpu.VMEM((gchunk,), jnp.int32),
            pltpu.VMEM((gchunk,), jnp.int32),
            pltpu.SemaphoreType.DMA,
            pltpu.SemaphoreType.DMA,
            pltpu.SemaphoreType.DMA,
            pltpu.SemaphoreType.DMA,
            pltpu.SemaphoreType.DMA,
        ],
    )
    def _sc_gather(t0_hbm, t1_hbm, i0_hbm, i1_hbm, out_hbm,
                   i0sA, i1sA, i0sB, i1sB,
                   w0A, w1A, w0B, w1B, bufo0, bufo1, bufs0, bufs1, idx0v, idx1v,
                   wsemA, wsemB, ssem0, ssem1, fsem):
        wid = lax.axis_index("s") * NC + lax.axis_index("c")
        base = wid * ew
        bufos = (bufo0, bufo1)
        ssems = (ssem0, ssem1)

        def issue(k, i0s, i1s, w0, w1, wsem):
            off = base + k * super_
            pltpu.sync_copy(i0_hbm.at[pl.ds(off, super_)], i0s.at[pl.ds(0, super_)])
            pltpu.sync_copy(i1_hbm.at[pl.ds(off, super_)], i1s.at[pl.ds(0, super_)])
            head0 = i0s[pl.ds(0, 16)]
            tail0 = i0s[pl.ds(super_ - 16, 16)]
            head1 = i1s[pl.ds(0, 16)]
            tail1 = i1s[pl.ds(super_ - 16, 16)]
            lo0, hi0 = head0[0], tail0[15]
            lo1, hi1 = head1[0], tail1[15]
            st0 = jnp.minimum((lo0 // 8) * 8, N_NODES - wmax)
            st1 = jnp.minimum((lo1 // 8) * 8, N_NODES - wmax)
            ok = jnp.logical_and(hi0 - st0 < wmax, hi1 - st1 < wmax)

            @pl.when(ok)
            def _():
                pltpu.async_copy(t0_hbm.at[pl.ds(pl.multiple_of(st0, 8), wmax)], w0, wsem)
                pltpu.async_copy(t1_hbm.at[pl.ds(pl.multiple_of(st1, 8), wmax)], w1, wsem)

            return ok.astype(jnp.int32), st0, st1

        def store(sub, off, p):
            pltpu.async_copy(bufos[p], out_hbm.at[pl.ds(off + sub * gchunk, gchunk)], ssems[p])

        def wait_store(p):
            pltpu.make_async_copy(bufos[p], out_hbm.at[pl.ds(base, gchunk)], ssems[p]).wait()

        def process(k, i0s, i1s, w0, w1, wsem, state):
            ok, st0, st1 = state
            off = base + k * super_

            def fast():
                pltpu.make_async_copy(t0_hbm.at[pl.ds(0, wmax)], w0, wsem).wait()
                pltpu.make_async_copy(t1_hbm.at[pl.ds(0, wmax)], w1, wsem).wait()
                for sub in range(nsub):
                    p = sub % 2
                    if sub >= 2:
                        wait_store(p)
                    bo = bufos[p]

                    def grp(g, c2, _sub=sub, _bo=bo):
                        e0 = _sub * gchunk + g * 8
                        r0v = i0s[pl.ds(e0, 16)] - st0
                        r1v = i1s[pl.ds(e0, 16)] - st1
                        for lane in range(8):
                            r0 = r0v[lane]
                            r1 = r1v[lane]
                            el = g * 8 + lane
                            va = [w0[r0, pl.ds(c * 16, 16)] for c in range(WSG // 16)]
                            vb = [w1[r1, pl.ds(c * 16, 16)] for c in range(WSG // 16)]
                            for c in range(WSG // 16):
                                _bo[el, pl.ds(c * 16, 16)] = va[c]
                                _bo[el, pl.ds(WSG + c * 16, 16)] = vb[c]
                        return c2

                    lax.fori_loop(0, gchunk // 8, grp, 0)
                    store(sub, off, p)
                wait_store(nsub % 2)
                wait_store((nsub - 1) % 2)

            def slow():
                def sub_body(sub, c3):
                    o = off + sub * gchunk
                    pltpu.sync_copy(i0_hbm.at[pl.ds(o, gchunk)], idx0v)
                    pltpu.sync_copy(i1_hbm.at[pl.ds(o, gchunk)], idx1v)
                    cpa = pltpu.async_copy(t0_hbm.at[idx0v], bufs0, fsem)
                    cpb = pltpu.async_copy(t1_hbm.at[idx1v], bufs1, fsem)
                    cpa.wait()
                    cpb.wait()

                    def row(r, c2):
                        for c in range(WSG // 16):
                            bufo0[r, pl.ds(c * 16, 16)] = bufs0[r, pl.ds(c * 16, 16)]
                            bufo0[r, pl.ds(WSG + c * 16, 16)] = bufs1[r, pl.ds(c * 16, 16)]
                        return c2

                    lax.fori_loop(0, gchunk, row, 0)
                    pltpu.sync_copy(bufo0, out_hbm.at[pl.ds(o, gchunk)])
                    return c3

                lax.fori_loop(0, nsub, sub_body, 0)

            lax.cond(ok == 1, fast, slow)

        stateA = issue(0, i0sA, i1sA, w0A, w1A, wsemA)

        def pair(i, stA):
            k0 = 2 * i
            k1 = 2 * i + 1
            stB = issue(k1, i0sB, i1sB, w0B, w1B, wsemB)
            process(k0, i0sA, i1sA, w0A, w1A, wsemA, stA)
            stA_next = lax.cond(
                k0 + 2 < n_super,
                lambda: issue(k0 + 2, i0sA, i1sA, w0A, w1A, wsemA),
                lambda: (jnp.int32(0), jnp.int32(0), jnp.int32(0)),
            )
            process(k1, i0sB, i1sB, w0B, w1B, wsemB, stB)
            return stA_next

        st_last = lax.fori_loop(0, n_super // 2, pair, stateA)
        process(n_super - 1, i0sA, i1sA, w0A, w1A, wsemA, st_last)

    return _sc_gather


def _make_sc_segsum(n_edges, chunk):
    ew = n_edges // NW
    n_chunks = ew // chunk
    assert ew % chunk == 0 and chunk % 8 == 0 and n_chunks >= 2

    @functools.partial(
        pl.kernel,
        out_type=jax.ShapeDtypeStruct((NC, N_NODES, D), jnp.float32),
        mesh=_SC_MESH,
        scratch_types=[
            pltpu.VMEM((chunk,), jnp.int32),
            pltpu.VMEM((chunk, D), jnp.float32),
            pltpu.VMEM((chunk,), jnp.int32),
            pltpu.VMEM((chunk, D), jnp.float32),
            pltpu.VMEM_SHARED((N_NODES, D), jnp.float32),
            pltpu.SemaphoreType.DMA,
            pltpu.SemaphoreType.DMA,
        ],
    )
    def _sc_segsum(t_hbm, i0_hbm, z_hbm, out_hbm, i0v0, tv0, i0v1, tv1,
                   agg_sh, lsem0, lsem1):
        cid = lax.axis_index("c")
        sid = lax.axis_index("s")
        wid = sid * NC + cid
        base = wid * ew

        pltpu.sync_copy(z_hbm.at[pl.ds(sid * ROWS_PER_TILE, ROWS_PER_TILE)],
                        agg_sh.at[pl.ds(sid * ROWS_PER_TILE, ROWS_PER_TILE)])

        @pl.when(sid == NS - 1)
        def _():
            pltpu.sync_copy(z_hbm.at[pl.ds(TAIL_ROW0, TAIL_ROWS)],
                            agg_sh.at[pl.ds(TAIL_ROW0, TAIL_ROWS)])

        plsc.subcore_barrier()

        def start(j, i0v, tv, lsem):
            off = base + j * chunk
            pltpu.async_copy(i0_hbm.at[pl.ds(off, chunk)], i0v, lsem)
            pltpu.async_copy(t_hbm.at[pl.ds(off, chunk)], tv, lsem)

        def wait_l(i0v, tv, lsem):
            pltpu.make_async_copy(i0_hbm.at[pl.ds(base, chunk)], i0v, lsem).wait()
            pltpu.make_async_copy(t_hbm.at[pl.ds(base, chunk)], tv, lsem).wait()

        start(0, i0v0, tv0, lsem0)
        start(1, i0v1, tv1, lsem1)

        def pair(i, carry):
            j0 = 2 * i
            j1 = 2 * i + 1
            wait_l(i0v0, tv0, lsem0)
            pltpu.sync_copy(tv0, agg_sh.at[i0v0], add=True)

            @pl.when(j0 + 2 < n_chunks)
            def _():
                start(j0 + 2, i0v0, tv0, lsem0)

            wait_l(i0v1, tv1, lsem1)
            pltpu.sync_copy(tv1, agg_sh.at[i0v1], add=True)

            @pl.when(j1 + 2 < n_chunks)
            def _():
                start(j1 + 2, i0v1, tv1, lsem1)

            return carry

        lax.fori_loop(0, n_chunks // 2, pair, 0)
        if n_chunks % 2 == 1:
            wait_l(i0v0, tv0, lsem0)
            pltpu.sync_copy(tv0, agg_sh.at[i0v0], add=True)
        plsc.subcore_barrier()
        pltpu.sync_copy(agg_sh.at[pl.ds(sid * ROWS_PER_TILE, ROWS_PER_TILE)],
                        out_hbm.at[cid, pl.ds(sid * ROWS_PER_TILE, ROWS_PER_TILE)])

        @pl.when(sid == NS - 1)
        def _():
            pltpu.sync_copy(agg_sh.at[pl.ds(TAIL_ROW0, TAIL_ROWS)],
                            out_hbm.at[cid, pl.ds(TAIL_ROW0, TAIL_ROWS)])

    return _sc_segsum


_sc_gather_half = _make_sc_gather(EH, 200, 40, 32)
_sc_segsum_half = _make_sc_segsum(EH, 40)


def kernel(atom_features, edges_sph_features, state_attrs, pair_indices,
           atom_graph_indices, bond_graph_indices, kernel_s, bias_s,
           kernel_g, bias_g):
    del state_attrs, atom_graph_indices, bond_graph_indices
    idx = pair_indices.astype(jnp.int32)
    i0a, i0b = idx[:EH, 0], idx[EH:, 0]
    i1a, i1b = idx[:EH, 1], idx[EH:, 1]
    ea, eb = edges_sph_features[:EH], edges_sph_features[EH:]
    w0 = jnp.concatenate([kernel_s[:D], kernel_g[:D]], axis=1)
    w1 = jnp.concatenate([kernel_s[D:2 * D], kernel_g[D:2 * D]], axis=1)
    we = jnp.concatenate([kernel_s[2 * D:], kernel_g[2 * D:]], axis=1)
    bias = jnp.concatenate([bias_s, bias_g]).reshape(1, SG)
    zeros = jnp.zeros((N_NODES, D), jnp.float32)

    def half_chains(t0, t1):
        sga = _sc_gather_half(t0, t1, i0a, i1a)
        ta = _edge_gate(sga, ea, we)
        sgb = _sc_gather_half(t0, t1, i0b, i1b)
        pa = _sc_segsum_half(ta, i0a, zeros)
        tb = _edge_gate(sgb, eb, we)
        pb = _sc_segsum_half(tb, i0b, zeros)
        return pa, pb

    af = atom_features
    t0, t1 = _tables(af, w0, w1, bias)
    pa, pb = half_chains(t0, t1)
    af, t0, t1 = _update_tables(af, pa, pb, w0, w1, bias)
    pa, pb = half_chains(t0, t1)
    return _final_update(af, pa, pb)
